# baseline (speedup 1.0000x reference)
import jax
import jax.numpy as jnp
from jax import lax
from jax.experimental import pallas as pl
from jax.experimental.pallas import tpu as pltpu
from jax.experimental.pallas import tpu_sc as plsc

_N = 2000
_NW = 16
_RPW = 128
_GPW = _RPW // 16

_EDGE_PAIRS = ((0, 3), (1, 0), (2, 1), (3, 2))


def _sqrt_f32(x):
    xi = plsc.bitcast(x, jnp.int32)
    y = plsc.bitcast(jnp.int32(0x5F3759DF) - (xi >> 1), jnp.float32)
    for _ in range(3):
        y = y * (1.5 - 0.5 * (x * y) * y)
    return x * y


def _body(pts_hbm, bbox_hbm, w_hbm, out_hbm,
          pts_v, bbox_v, w_v, acc_v, shared, red_v, out_v,
          sem_p, sem_b, sem_w):
    wid = lax.axis_index("s")
    f32 = jnp.float32

    base = wid * _RPW
    arow = jnp.minimum(base, _N - _RPW)
    drows = base - arow
    nvalid = jnp.minimum(_RPW, _N - base)

    cp_p = pltpu.async_copy(pts_hbm.at[pl.ds(arow * 18, _RPW * 18)], pts_v, sem_p)
    cp_b = pltpu.async_copy(bbox_hbm.at[pl.ds(arow * 8, _RPW * 8)], bbox_v, sem_b)
    cp_w = pltpu.async_copy(w_hbm.at[pl.ds(arow, _RPW)], w_v, sem_w)
    cp_p.wait()
    cp_b.wait()
    cp_w.wait()

    lane = lax.iota(jnp.int32, 16)
    zero16 = jnp.zeros((16,), f32)

    @plsc.parallel_loop(0, _GPW, unroll=8, carry=(zero16, zero16, zero16))
    def group(i, carry):
        sum_d, cnt, wcnt = carry
        local_row = i * 16 + lane
        valid = (local_row < nvalid).astype(f32)
        buf_row = drows + jnp.minimum(local_row, nvalid - 1)

        b_base = buf_row * 8
        xs = [plsc.load_gather(bbox_v, [b_base + 2 * v]) for v in range(4)]
        ys = [plsc.load_gather(bbox_v, [b_base + 2 * v + 1]) for v in range(4)]
        cx = (xs[0] + xs[2]) * 0.5
        cy = (ys[0] + ys[2]) * 0.5

        p_base = buf_row * 18
        for k in range(9):
            px = plsc.load_gather(pts_v, [p_base + 2 * k])
            py = plsc.load_gather(pts_v, [p_base + 2 * k + 1])

            inside = None
            for vi, vj in _EDGE_PAIRS:
                xi, yi = xs[vi], ys[vi]
                xj, yj = xs[vj], ys[vj]
                cond1 = (yi > py) != (yj > py)
                dy = yj - yi
                safe_dy = jnp.where(dy == 0.0, f32(1.0), dy)
                x_cross = (xj - xi) * (py - yi) / safe_dy + xi
                crossing = cond1 & (px < x_cross)
                inside = crossing if inside is None else (inside ^ crossing)

            outside = jnp.where(inside, zero16, valid)
            dx = px - cx
            dyc = py - cy
            dist = 0.2 * _sqrt_f32(dx * dx + dyc * dyc)
            sum_d = sum_d + dist * outside
            cnt = cnt + outside

        wv = w_v[pl.ds(drows + i * 16, 16)]
        wcnt = wcnt + jnp.where(wv > 0.0, valid, zero16)
        return sum_d, cnt, wcnt

    sum_d, cnt, wcnt = group

    acc_v[pl.ds(0, 16)] = sum_d
    acc_v[pl.ds(16, 16)] = cnt
    acc_v[pl.ds(32, 16)] = wcnt
    pltpu.sync_copy(acc_v, shared.at[pl.ds(wid * 48, 48)])
    plsc.subcore_barrier()

    @pl.when(wid == 0)
    def _finalize():
        pltpu.sync_copy(shared, red_v)
        tot_d = zero16
        tot_c = zero16
        tot_w = zero16
        for w in range(_NW):
            tot_d = tot_d + red_v[pl.ds(w * 48, 16)]
            tot_c = tot_c + red_v[pl.ds(w * 48 + 16, 16)]
            tot_w = tot_w + red_v[pl.ds(w * 48 + 32, 16)]
        s = plsc.cumsum(tot_d)
        c = plsc.cumsum(tot_c)
        avg = plsc.cumsum(tot_w) + f32(1e-6)
        loss = jnp.where(c > 0.0, s / jnp.maximum(c, f32(1.0)), zero16) / avg
        out_v[...] = lax.rev(loss, (0,))
        pltpu.sync_copy(out_v.at[pl.ds(0, 1)], out_hbm)


@jax.jit
def _run(pts_flat, bbox_flat, w_arr):
    mesh = plsc.VectorSubcoreMesh(
        core_axis_name="c", subcore_axis_name="s", num_cores=1)
    return pl.kernel(
        _body,
        mesh=mesh,
        compiler_params=pltpu.CompilerParams(needs_layout_passes=False),
        out_type=jax.ShapeDtypeStruct((1,), jnp.float32),
        scratch_types=[
            pltpu.VMEM((_RPW * 18,), jnp.float32),
            pltpu.VMEM((_RPW * 8,), jnp.float32),
            pltpu.VMEM((_RPW,), jnp.float32),
            pltpu.VMEM((48,), jnp.float32),
            pltpu.VMEM_SHARED((_NW * 48,), jnp.float32),
            pltpu.VMEM((_NW * 48,), jnp.float32),
            pltpu.VMEM((16,), jnp.float32),
            pltpu.SemaphoreType.DMA,
            pltpu.SemaphoreType.DMA,
            pltpu.SemaphoreType.DMA,
        ],
    )(pts_flat, bbox_flat, w_arr)


def kernel(pts, gt_bboxes, weight):
    return _run(pts.reshape(-1), gt_bboxes.reshape(-1), weight)


# speedup vs baseline: 1.0959x; 1.0959x over previous
import jax
import jax.numpy as jnp
from jax import lax
from jax.experimental import pallas as pl
from jax.experimental.pallas import tpu as pltpu
from jax.experimental.pallas import tpu_sc as plsc

_N = 2000
_NW = 16
_RPW = 128
_GPW = _RPW // 16

_EDGE_PAIRS = ((0, 3), (1, 0), (2, 1), (3, 2))


def _sqrt_f32(x):
    xi = plsc.bitcast(x, jnp.int32)
    y = plsc.bitcast(jnp.int32(0x5F3759DF) - (xi >> 1), jnp.float32)
    for _ in range(3):
        y = y * (1.5 - 0.5 * (x * y) * y)
    return x * y


def _body(pts_hbm, bbox_hbm, w_hbm, out_hbm,
          pts_v, bbox_v, w_v, acc_v, shared, red_v, out_v,
          sem_p, sem_b, sem_w):
    wid = lax.axis_index("s")
    f32 = jnp.float32

    base = wid * _RPW
    arow = jnp.minimum(base, _N - _RPW)
    drows = base - arow
    nvalid = jnp.minimum(_RPW, _N - base)

    cp_p = pltpu.async_copy(pts_hbm.at[pl.ds(arow * 18, _RPW * 18)], pts_v, sem_p)
    cp_b = pltpu.async_copy(bbox_hbm.at[pl.ds(arow * 8, _RPW * 8)], bbox_v, sem_b)
    cp_w = pltpu.async_copy(w_hbm.at[pl.ds(arow, _RPW)], w_v, sem_w)
    cp_p.wait()
    cp_b.wait()
    cp_w.wait()

    lane = lax.iota(jnp.int32, 16)
    zero16 = jnp.zeros((16,), f32)

    @plsc.parallel_loop(0, _GPW, unroll=2, carry=(zero16, zero16, zero16))
    def group(i, carry):
        sum_d, cnt, wcnt = carry
        local_row = i * 16 + lane
        valid = (local_row < nvalid).astype(f32)
        buf_row = drows + jnp.minimum(local_row, nvalid - 1)

        b_base = buf_row * 8
        xs = [plsc.load_gather(bbox_v, [b_base + 2 * v]) for v in range(4)]
        ys = [plsc.load_gather(bbox_v, [b_base + 2 * v + 1]) for v in range(4)]
        cx = (xs[0] + xs[2]) * 0.5
        cy = (ys[0] + ys[2]) * 0.5

        edge_c = []
        for vi, vj in _EDGE_PAIRS:
            dy = ys[vj] - ys[vi]
            safe_dy = jnp.where(dy == 0.0, f32(1.0), dy)
            edge_c.append((xs[vi], ys[vi], xs[vj] - xs[vi], safe_dy))

        p_base = buf_row * 18
        for k in range(9):
            px = plsc.load_gather(pts_v, [p_base + 2 * k])
            py = plsc.load_gather(pts_v, [p_base + 2 * k + 1])

            inside = None
            for e, (vi, vj) in enumerate(_EDGE_PAIRS):
                xi, yi, dxji, safe_dy = edge_c[e]
                yj = ys[vj]
                cond1 = (yi > py) != (yj > py)
                x_cross = dxji * (py - yi) / safe_dy + xi
                crossing = cond1 & (px < x_cross)
                inside = crossing if inside is None else (inside ^ crossing)

            outside = jnp.where(inside, zero16, valid)
            dx = px - cx
            dyc = py - cy
            dist = 0.2 * _sqrt_f32(dx * dx + dyc * dyc)
            sum_d = sum_d + dist * outside
            cnt = cnt + outside

        wv = w_v[pl.ds(drows + i * 16, 16)]
        wcnt = wcnt + jnp.where(wv > 0.0, valid, zero16)
        return sum_d, cnt, wcnt

    sum_d, cnt, wcnt = group

    acc_v[pl.ds(0, 16)] = sum_d
    acc_v[pl.ds(16, 16)] = cnt
    acc_v[pl.ds(32, 16)] = wcnt
    pltpu.sync_copy(acc_v, shared.at[pl.ds(wid * 48, 48)])
    plsc.subcore_barrier()

    @pl.when(wid == 0)
    def _finalize():
        pltpu.sync_copy(shared, red_v)
        tot_d = zero16
        tot_c = zero16
        tot_w = zero16
        for w in range(_NW):
            tot_d = tot_d + red_v[pl.ds(w * 48, 16)]
            tot_c = tot_c + red_v[pl.ds(w * 48 + 16, 16)]
            tot_w = tot_w + red_v[pl.ds(w * 48 + 32, 16)]
        s = plsc.cumsum(tot_d)
        c = plsc.cumsum(tot_c)
        avg = plsc.cumsum(tot_w) + f32(1e-6)
        loss = jnp.where(c > 0.0, s / jnp.maximum(c, f32(1.0)), zero16) / avg
        out_v[...] = lax.rev(loss, (0,))
        pltpu.sync_copy(out_v.at[pl.ds(0, 1)], out_hbm)


@jax.jit
def _run(pts_flat, bbox_flat, w_arr):
    mesh = plsc.VectorSubcoreMesh(
        core_axis_name="c", subcore_axis_name="s", num_cores=1)
    return pl.kernel(
        _body,
        mesh=mesh,
        compiler_params=pltpu.CompilerParams(needs_layout_passes=False),
        out_type=jax.ShapeDtypeStruct((1,), jnp.float32),
        scratch_types=[
            pltpu.VMEM((_RPW * 18,), jnp.float32),
            pltpu.VMEM((_RPW * 8,), jnp.float32),
            pltpu.VMEM((_RPW,), jnp.float32),
            pltpu.VMEM((48,), jnp.float32),
            pltpu.VMEM_SHARED((_NW * 48,), jnp.float32),
            pltpu.VMEM((_NW * 48,), jnp.float32),
            pltpu.VMEM((16,), jnp.float32),
            pltpu.SemaphoreType.DMA,
            pltpu.SemaphoreType.DMA,
            pltpu.SemaphoreType.DMA,
        ],
    )(pts_flat, bbox_flat, w_arr)


def kernel(pts, gt_bboxes, weight):
    return _run(pts.reshape(-1), gt_bboxes.reshape(-1), weight)
